# baseline (speedup 1.0000x reference)
import functools
import numpy as np
import jax
import jax.numpy as jnp
from jax import lax
from jax.experimental import pallas as pl
from jax.experimental.pallas import tpu as pltpu
from jax.experimental.pallas import tpu_sc as plsc

D = 768
MAX_LEN = 512
NLANE = 16
NSLICE = D // NLANE
C = 16
NBUF = 4
NRES = 2
NW = 32
NSEG = 3


def _pe_table():
    position = np.arange(0, MAX_LEN, dtype=np.float32)[:, None]
    div_term = np.exp(
        np.arange(0, D, 2, dtype=np.float32) * -(np.log(10000.0) / D)
    )
    pe = np.zeros((MAX_LEN, D), dtype=np.float32)
    pe[:, 0::2] = np.sin(position * div_term)
    pe[:, 1::2] = np.cos(position * div_term)
    return pe


def _comb_kernel(pe_ref, seg_ref, out_ref):
    pe = pe_ref[...]
    for s in range(NSEG):
        out_ref[pl.ds(s * MAX_LEN, MAX_LEN), :] = pe + seg_ref[pl.ds(s, 1), :]


def _build_comb(pe, segment_table):
    return pl.pallas_call(
        _comb_kernel,
        out_shape=jax.ShapeDtypeStruct((NSEG * MAX_LEN, D), jnp.float32),
    )(pe, segment_table)


@functools.lru_cache(maxsize=None)
def _make_kernel(B, L):
    TOK = B * L
    rows_per_w = B // NW
    n_lc = L // C
    mesh = plsc.VectorSubcoreMesh(
        core_axis_name="c", subcore_axis_name="s", num_cores=2, num_subcores=16
    )

    @functools.partial(
        pl.kernel,
        out_type=jax.ShapeDtypeStruct((TOK, D), jnp.float32),
        mesh=mesh,
        scratch_types=[
            pltpu.VMEM((NBUF, C, D), jnp.float32),
            pltpu.VMEM((NBUF, C, D), jnp.float32),
            pltpu.VMEM((NRES, C, D), jnp.float32),
            pltpu.VMEM((rows_per_w, C), jnp.int32),
            pltpu.VMEM((rows_per_w, C), jnp.int32),
            pltpu.VMEM((rows_per_w, C), jnp.int32),
            pltpu.SemaphoreType.DMA,
            pltpu.SemaphoreType.DMA,
            pltpu.SemaphoreType.DMA,
            pltpu.SemaphoreType.DMA,
            pltpu.SemaphoreType.DMA,
            pltpu.SemaphoreType.DMA,
            pltpu.SemaphoreType.DMA,
            pltpu.SemaphoreType.DMA,
            pltpu.SemaphoreType.DMA,
            pltpu.SemaphoreType.DMA,
            pltpu.SemaphoreType.DMA,
        ],
    )
    def emb_kernel(x_hbm, seg_hbm, tok_tab, comb_hbm, out_hbm,
                   tok_v, cmb_v, res_v, idx_v, sidx_v, cidx_v,
                   sem_st, sem_t0, sem_t1, sem_t2, sem_t3,
                   sem_c0, sem_c1, sem_c2, sem_c3, sem_o0, sem_o1):
        wid = lax.axis_index("s") * 2 + lax.axis_index("c")
        row0 = wid * rows_per_w
        sem_t = (sem_t0, sem_t1, sem_t2, sem_t3)
        sem_c = (sem_c0, sem_c1, sem_c2, sem_c3)
        sem_o = (sem_o0, sem_o1)
        lane = lax.iota(jnp.int32, NLANE)

        def valu_add(buf, rb):
            def body(i, carry):
                for c in range(NSLICE):
                    sl = pl.ds(c * NLANE, NLANE)
                    res_v[rb, i, sl] = tok_v[buf, i, sl] + cmb_v[buf, i, sl]
                return carry
            lax.fori_loop(0, C, body, None)

        def lchunk(lc, carry):
            l0 = lc * C
            sts = []
            for p in range(rows_per_w):
                base = (row0 + p) * L + l0
                sts.append(pltpu.async_copy(
                    x_hbm.at[pl.ds(base, C)], idx_v.at[p], sem_st))
                sts.append(pltpu.async_copy(
                    seg_hbm.at[pl.ds(base, C)], sidx_v.at[p], sem_st))
            for d in sts:
                d.wait()
            for p in range(rows_per_w):
                for j in range(C // NLANE):
                    sl = pl.ds(j * NLANE, NLANE)
                    cidx_v[p, sl] = (
                        sidx_v[p, sl] * L + (l0 + j * NLANE) + lane
                    )

            g_t = {}
            g_c = {}
            o = {}

            H = C // 2

            def launch(q):
                buf = q % NBUF
                g_t[q] = tuple(
                    pltpu.async_copy(
                        tok_tab.at[idx_v.at[q, pl.ds(h * H, H)]],
                        tok_v.at[buf, pl.ds(h * H, H)], sem_t[buf])
                    for h in range(2))
                g_c[q] = tuple(
                    pltpu.async_copy(
                        comb_hbm.at[cidx_v.at[q, pl.ds(h * H, H)]],
                        cmb_v.at[buf, pl.ds(h * H, H)], sem_c[buf])
                    for h in range(2))

            for q in range(min(NBUF, rows_per_w)):
                launch(q)
            for p in range(rows_per_w):
                buf = p % NBUF
                rb = p % NRES
                for d in g_t[p] + g_c[p]:
                    d.wait()
                if p >= NRES:
                    o[p - NRES].wait()
                valu_add(buf, rb)
                o[p] = pltpu.async_copy(
                    res_v.at[rb], out_hbm.at[pl.ds((row0 + p) * L + l0, C)],
                    sem_o[rb])
                q = p + NBUF
                if q < rows_per_w:
                    launch(q)
            for p in range(rows_per_w - NRES, rows_per_w):
                o[p].wait()
            return carry

        lax.fori_loop(0, n_lc, lchunk, None)

    return emb_kernel


def kernel(x, segment_label, token_table, segment_table):
    B, L = x.shape
    x_i32 = x.reshape(-1).astype(jnp.int32)
    s_i32 = segment_label.reshape(-1).astype(jnp.int32)
    pe = jnp.asarray(_pe_table()[:L])
    comb = _build_comb(pe, segment_table)
    out = _make_kernel(B, L)(x_i32, s_i32, token_table, comb)
    return out.reshape(B, L, D)


# speedup vs baseline: 1.5466x; 1.5466x over previous
import functools
import numpy as np
import jax
import jax.numpy as jnp
from jax import lax
from jax.experimental import pallas as pl
from jax.experimental.pallas import tpu as pltpu
from jax.experimental.pallas import tpu_sc as plsc

D = 768
MAX_LEN = 512
NLANE = 16
NSLICE = D // NLANE
C = 16
NBUF = 3
NRES = 3
NW = 32
NSEG = 3


def _pe_table():
    position = np.arange(0, MAX_LEN, dtype=np.float32)[:, None]
    div_term = np.exp(
        np.arange(0, D, 2, dtype=np.float32) * -(np.log(10000.0) / D)
    )
    pe = np.zeros((MAX_LEN, D), dtype=np.float32)
    pe[:, 0::2] = np.sin(position * div_term)
    pe[:, 1::2] = np.cos(position * div_term)
    return pe


def _comb_kernel(pe_ref, seg_ref, out_ref):
    pe = pe_ref[...]
    for s in range(NSEG):
        out_ref[pl.ds(s * MAX_LEN, MAX_LEN), :] = pe + seg_ref[pl.ds(s, 1), :]


def _build_comb(pe, segment_table):
    return pl.pallas_call(
        _comb_kernel,
        out_shape=jax.ShapeDtypeStruct((NSEG * MAX_LEN, D), jnp.float32),
    )(pe, segment_table)


@functools.lru_cache(maxsize=None)
def _make_kernel(B, L):
    TOK = B * L
    rows_per_w = B // NW
    n_lc = L // C
    mesh = plsc.VectorSubcoreMesh(
        core_axis_name="c", subcore_axis_name="s", num_cores=2, num_subcores=16
    )

    @functools.partial(
        pl.kernel,
        out_type=jax.ShapeDtypeStruct((TOK, D), jnp.float32),
        mesh=mesh,
        scratch_types=[
            pltpu.VMEM((NBUF, C, D), jnp.float32),
            pltpu.VMEM((NBUF, C, D), jnp.float32),
            pltpu.VMEM((NRES, C, D), jnp.float32),
            pltpu.VMEM((rows_per_w, C), jnp.int32),
            pltpu.VMEM((rows_per_w, C), jnp.int32),
            pltpu.VMEM((rows_per_w, C), jnp.int32),
            pltpu.SemaphoreType.DMA,
            pltpu.SemaphoreType.DMA,
            pltpu.SemaphoreType.DMA,
            pltpu.SemaphoreType.DMA,
            pltpu.SemaphoreType.DMA,
            pltpu.SemaphoreType.DMA,
            pltpu.SemaphoreType.DMA,
            pltpu.SemaphoreType.DMA,
            pltpu.SemaphoreType.DMA,
            pltpu.SemaphoreType.DMA,
        ],
    )
    def emb_kernel(x_hbm, seg_hbm, tok_tab, comb_hbm, out_hbm,
                   tok_v, cmb_v, res_v, idx_v, sidx_v, cidx_v,
                   sem_st, sem_t0, sem_t1, sem_t2,
                   sem_c0, sem_c1, sem_c2, sem_o0, sem_o1, sem_o2):
        wid = lax.axis_index("s") * 2 + lax.axis_index("c")
        row0 = wid * rows_per_w
        sem_t = (sem_t0, sem_t1, sem_t2)
        sem_c = (sem_c0, sem_c1, sem_c2)
        sem_o = (sem_o0, sem_o1, sem_o2)
        lane = lax.iota(jnp.int32, NLANE)

        def valu_add(buf, rb):
            @functools.partial(plsc.parallel_loop, 0, C, unroll=2)
            def body(i):
                for c in range(NSLICE):
                    sl = pl.ds(c * NLANE, NLANE)
                    res_v[rb, i, sl] = tok_v[buf, i, sl] + cmb_v[buf, i, sl]

        def lchunk(lc, carry):
            l0 = lc * C
            sts = []
            for p in range(rows_per_w):
                base = (row0 + p) * L + l0
                sts.append(pltpu.async_copy(
                    x_hbm.at[pl.ds(base, C)], idx_v.at[p], sem_st))
                sts.append(pltpu.async_copy(
                    seg_hbm.at[pl.ds(base, C)], sidx_v.at[p], sem_st))
            for d in sts:
                d.wait()
            for p in range(rows_per_w):
                for j in range(C // NLANE):
                    sl = pl.ds(j * NLANE, NLANE)
                    cidx_v[p, sl] = (
                        sidx_v[p, sl] * L + (l0 + j * NLANE) + lane
                    )

            g_t = {}
            g_c = {}
            o = {}

            def launch(q):
                buf = q % NBUF
                g_t[q] = (pltpu.async_copy(tok_tab.at[idx_v.at[q]],
                                           tok_v.at[buf], sem_t[buf]),)
                g_c[q] = (pltpu.async_copy(comb_hbm.at[cidx_v.at[q]],
                                           cmb_v.at[buf], sem_c[buf]),)

            for q in range(min(NBUF, rows_per_w)):
                launch(q)
            for p in range(rows_per_w):
                buf = p % NBUF
                rb = p % NRES
                for d in g_t[p] + g_c[p]:
                    d.wait()
                if p >= NRES:
                    o[p - NRES].wait()
                valu_add(buf, rb)
                o[p] = pltpu.async_copy(
                    res_v.at[rb], out_hbm.at[pl.ds((row0 + p) * L + l0, C)],
                    sem_o[rb])
                q = p + NBUF
                if q < rows_per_w:
                    launch(q)
            for p in range(rows_per_w - NRES, rows_per_w):
                o[p].wait()
            return carry

        lax.fori_loop(0, n_lc, lchunk, None)

    return emb_kernel


def kernel(x, segment_label, token_table, segment_table):
    B, L = x.shape
    x_i32 = x.reshape(-1).astype(jnp.int32)
    s_i32 = segment_label.reshape(-1).astype(jnp.int32)
    pe = jnp.asarray(_pe_table()[:L])
    comb = _build_comb(pe, segment_table)
    out = _make_kernel(B, L)(x_i32, s_i32, token_table, comb)
    return out.reshape(B, L, D)
